# baseline (speedup 1.0000x reference)
import functools

import jax
import jax.numpy as jnp
from jax import lax
from jax.experimental import pallas as pl
from jax.experimental.pallas import tpu as pltpu
from jax.experimental.pallas import tpu_sc as plsc

VOCAB = 100000
DIM = 2048
TOTAL = 4 * 4096

NC, NS = 2, 16
NW = NC * NS
BPW = TOTAL // NW
CHUNK = 8
NBUF = 4
NCH = BPW // CHUNK
NGRP = NCH // NBUF


def _emb_kernel(ids_hbm, table_hbm, out_hbm, idx_v, bufs, *sems):
    gsems = list(sems[:NBUF])
    ssems = list(sems[NBUF:])
    wid = lax.axis_index("s") * NC + lax.axis_index("c")
    base = wid * BPW

    pltpu.sync_copy(ids_hbm.at[pl.ds(base, BPW)], idx_v)

    def start_gather(j, b):
        off = pl.multiple_of(j * CHUNK, 8)
        pltpu.async_copy(
            table_hbm.at[idx_v.at[pl.ds(off, CHUNK)]], bufs.at[b], gsems[b])

    def wait_gather(b):
        pltpu.make_async_copy(
            table_hbm.at[pl.ds(0, CHUNK)], bufs.at[b], gsems[b]).wait()

    def start_scatter(j, b):
        row = pl.multiple_of(base + j * CHUNK, 8)
        pltpu.async_copy(bufs.at[b], out_hbm.at[pl.ds(row, CHUNK)], ssems[b])

    def wait_scatter(b):
        pltpu.make_async_copy(
            bufs.at[b], out_hbm.at[pl.ds(0, CHUNK)], ssems[b]).wait()

    for b in range(NBUF):
        start_gather(b, b)

    def group(g, carry):
        for b in range(NBUF):
            j = g * NBUF + b
            wait_gather(b)
            start_scatter(j, b)
            wait_scatter(b)
            start_gather(j + NBUF, b)
        return carry

    lax.fori_loop(0, NGRP - 1, group, 0)

    for b in range(NBUF):
        j = (NGRP - 1) * NBUF + b
        wait_gather(b)
        start_scatter(j, b)
        wait_scatter(b)


KTC = 16


def _tc_body(idx_ref, *refs):
    ins, out = refs[:KTC], refs[KTC]
    for k in range(KTC):
        out[k, :] = ins[k][0, 0, :]


def _tc_gather(ids, table, n_rows):
    table3 = table.reshape(VOCAB, 1, DIM)
    grid = (n_rows // KTC,)
    in_specs = [
        pl.BlockSpec((1, 1, DIM), (lambda i, idx_ref, k=k: (idx_ref[i * KTC + k], 0, 0)))
        for k in range(KTC)
    ]
    return pl.pallas_call(
        _tc_body,
        grid_spec=pltpu.PrefetchScalarGridSpec(
            num_scalar_prefetch=1,
            grid=grid,
            in_specs=in_specs,
            out_specs=pl.BlockSpec((KTC, DIM), lambda i, idx_ref: (i, 0)),
        ),
        out_shape=jax.ShapeDtypeStruct((n_rows, DIM), jnp.float32),
    )(ids, *([table3] * KTC))


@jax.jit
def _lookup_tc(ids_flat, table):
    return _tc_gather(ids_flat, table, TOTAL)


@jax.jit
def _lookup(ids_flat, table):
    mesh = plsc.VectorSubcoreMesh(core_axis_name="c", subcore_axis_name="s")
    fn = functools.partial(
        pl.kernel,
        out_type=jax.ShapeDtypeStruct((TOTAL, DIM), jnp.float32),
        mesh=mesh,
        scratch_types=[
            pltpu.VMEM((BPW,), jnp.int32),
            pltpu.VMEM((NBUF, CHUNK, DIM), jnp.float32),
        ] + [pltpu.SemaphoreType.DMA] * (2 * NBUF),
    )(_emb_kernel)
    return fn(ids_flat, table)


def kernel(input_ids, token_embeddings):
    ids_flat = input_ids.reshape(-1).astype(jnp.int32)
    out = _lookup_tc(ids_flat, token_embeddings)
    return out.reshape(input_ids.shape + (token_embeddings.shape[1],))


# speedup vs baseline: 1.4839x; 1.4839x over previous
import functools

import jax
import jax.numpy as jnp
from jax import lax
from jax.experimental import pallas as pl
from jax.experimental.pallas import tpu as pltpu
from jax.experimental.pallas import tpu_sc as plsc

VOCAB = 100000
DIM = 2048
TOTAL = 4 * 4096

N_TC = 1024
N_SC = TOTAL - N_TC

NC, NS = 2, 16
NW = NC * NS
BPW = N_SC // NW
CHUNK = 16
NBUF = 2
NCH = BPW // CHUNK
NGRP = NCH // NBUF


def _emb_kernel(ids_hbm, table_hbm, out_hbm, idx_v, bufs, *sems):
    gsems = list(sems[:NBUF])
    ssems = list(sems[NBUF:])
    wid = lax.axis_index("s") * NC + lax.axis_index("c")
    base = wid * BPW

    pltpu.sync_copy(ids_hbm.at[pl.ds(base, BPW)], idx_v)

    def start_gather(j, b):
        off = pl.multiple_of(j * CHUNK, 8)
        pltpu.async_copy(
            table_hbm.at[idx_v.at[pl.ds(off, CHUNK)]], bufs.at[b], gsems[b])

    def wait_gather(b):
        pltpu.make_async_copy(
            table_hbm.at[pl.ds(0, CHUNK)], bufs.at[b], gsems[b]).wait()

    def start_scatter(j, b):
        row = pl.multiple_of(base + j * CHUNK, 8)
        pltpu.async_copy(bufs.at[b], out_hbm.at[pl.ds(row, CHUNK)], ssems[b])

    def wait_scatter(b):
        pltpu.make_async_copy(
            bufs.at[b], out_hbm.at[pl.ds(0, CHUNK)], ssems[b]).wait()

    for b in range(NBUF):
        start_gather(b, b)

    def group(g, carry):
        for b in range(NBUF):
            j = g * NBUF + b
            wait_gather(b)
            start_scatter(j, b)
            wait_scatter(b)
            start_gather(j + NBUF, b)
        return carry

    lax.fori_loop(0, NGRP - 1, group, 0)

    for b in range(NBUF):
        j = (NGRP - 1) * NBUF + b
        wait_gather(b)
        start_scatter(j, b)
        wait_scatter(b)


def _sc_gather(ids, table, n_rows):
    mesh = plsc.VectorSubcoreMesh(core_axis_name="c", subcore_axis_name="s")
    fn = functools.partial(
        pl.kernel,
        out_type=jax.ShapeDtypeStruct((n_rows, DIM), jnp.float32),
        mesh=mesh,
        scratch_types=[
            pltpu.VMEM((BPW,), jnp.int32),
            pltpu.VMEM((NBUF, CHUNK, DIM), jnp.float32),
        ] + [pltpu.SemaphoreType.DMA] * (2 * NBUF),
    )(_emb_kernel)
    return fn(ids, table)


KTC = 16


def _tc_body(idx_ref, *refs):
    ins, out = refs[:KTC], refs[KTC]
    for k in range(KTC):
        out[k, :] = ins[k][0, 0, :]


def _tc_gather(ids, table, n_rows):
    table3 = table.reshape(VOCAB, 1, DIM)
    grid = (n_rows // KTC,)
    in_specs = [
        pl.BlockSpec((1, 1, DIM), (lambda i, idx_ref, k=k: (idx_ref[i * KTC + k], 0, 0)))
        for k in range(KTC)
    ]
    return pl.pallas_call(
        _tc_body,
        grid_spec=pltpu.PrefetchScalarGridSpec(
            num_scalar_prefetch=1,
            grid=grid,
            in_specs=in_specs,
            out_specs=pl.BlockSpec((KTC, DIM), lambda i, idx_ref: (i, 0)),
        ),
        out_shape=jax.ShapeDtypeStruct((n_rows, DIM), jnp.float32),
    )(ids, *([table3] * KTC))


@jax.jit
def _lookup(ids_flat, table):
    out_tc = _tc_gather(ids_flat[N_SC:], table, N_TC)
    out_sc = _sc_gather(ids_flat[:N_SC], table, N_SC)
    return jnp.concatenate([out_sc, out_tc], axis=0)


def kernel(input_ids, token_embeddings):
    ids_flat = input_ids.reshape(-1).astype(jnp.int32)
    out = _lookup(ids_flat, token_embeddings)
    return out.reshape(input_ids.shape + (token_embeddings.shape[1],))


# speedup vs baseline: 10.3564x; 6.9793x over previous
import functools

import jax
import jax.numpy as jnp
from jax import lax
from jax.experimental import pallas as pl
from jax.experimental.pallas import tpu as pltpu
from jax.experimental.pallas import tpu_sc as plsc

VOCAB = 100000
DIM = 2048
TOTAL = 4 * 4096

N_TC = 1024
N_SC = TOTAL - N_TC

NC, NS = 2, 16
NW = NC * NS
BPW = N_SC // NW
CHUNK = 16
NBUF = 2
NCH = BPW // CHUNK
NGRP = NCH // NBUF


def _emb_kernel(ids_hbm, table_hbm, out_hbm, idx_v, bufs, *sems):
    gsems = list(sems[:NBUF])
    ssems = list(sems[NBUF:])
    wid = lax.axis_index("s") * NC + lax.axis_index("c")
    base = wid * BPW

    pltpu.sync_copy(ids_hbm.at[pl.ds(base, BPW)], idx_v)

    def start_gather(j, b):
        off = pl.multiple_of(j * CHUNK, 8)
        pltpu.async_copy(
            table_hbm.at[idx_v.at[pl.ds(off, CHUNK)]], bufs.at[b], gsems[b])

    def wait_gather(b):
        pltpu.make_async_copy(
            table_hbm.at[pl.ds(0, CHUNK)], bufs.at[b], gsems[b]).wait()

    def start_scatter(j, b):
        row = pl.multiple_of(base + j * CHUNK, 8)
        pltpu.async_copy(bufs.at[b], out_hbm.at[pl.ds(row, CHUNK)], ssems[b])

    def wait_scatter(b):
        pltpu.make_async_copy(
            bufs.at[b], out_hbm.at[pl.ds(0, CHUNK)], ssems[b]).wait()

    for b in range(NBUF):
        start_gather(b, b)

    def group(g, carry):
        for b in range(NBUF):
            j = g * NBUF + b
            wait_gather(b)
            start_scatter(j, b)
            wait_scatter(b)
            start_gather(j + NBUF, b)
        return carry

    lax.fori_loop(0, NGRP - 1, group, 0)

    for b in range(NBUF):
        j = (NGRP - 1) * NBUF + b
        wait_gather(b)
        start_scatter(j, b)
        wait_scatter(b)


def _sc_gather(ids, table, n_rows):
    mesh = plsc.VectorSubcoreMesh(core_axis_name="c", subcore_axis_name="s")
    fn = functools.partial(
        pl.kernel,
        out_type=jax.ShapeDtypeStruct((n_rows, DIM), jnp.float32),
        mesh=mesh,
        scratch_types=[
            pltpu.VMEM((BPW,), jnp.int32),
            pltpu.VMEM((NBUF, CHUNK, DIM), jnp.float32),
        ] + [pltpu.SemaphoreType.DMA] * (2 * NBUF),
    )(_emb_kernel)
    return fn(ids, table)


KTC = 16


def _tc_body(idx_ref, table_hbm, out_ref, buf, s0, s1):
    m = pl.program_id(0)
    nm = pl.num_programs(0)
    sems = [s0, s1]

    def issue(step, slot):
        for k in range(KTC):
            idx = idx_ref[step * KTC + k]
            pltpu.make_async_copy(
                table_hbm.at[pl.ds(idx, 1)],
                buf.at[slot, pl.ds(k, 1)], sems[slot]).start()

    def drain(slot):
        for _ in range(KTC):
            pltpu.make_async_copy(
                table_hbm.at[pl.ds(0, 1)],
                buf.at[slot, pl.ds(0, 1)], sems[slot]).wait()

    @pl.when(m == 0)
    def _():
        issue(0, 0)

    issue(2 * m + 1, 1)
    drain(0)
    out_ref[0:KTC, :] = buf[0]

    @pl.when(m + 1 < nm)
    def _():
        issue(2 * m + 2, 0)

    drain(1)
    out_ref[KTC : 2 * KTC, :] = buf[1]


def _tc_gather(ids, table, n_rows):
    grid = (n_rows // (2 * KTC),)
    return pl.pallas_call(
        _tc_body,
        grid_spec=pltpu.PrefetchScalarGridSpec(
            num_scalar_prefetch=1,
            grid=grid,
            in_specs=[pl.BlockSpec(memory_space=pltpu.MemorySpace.HBM)],
            out_specs=pl.BlockSpec((2 * KTC, DIM), lambda i, idx_ref: (i, 0)),
            scratch_shapes=[
                pltpu.VMEM((2, KTC, DIM), jnp.float32),
                pltpu.SemaphoreType.DMA,
                pltpu.SemaphoreType.DMA,
            ],
        ),
        out_shape=jax.ShapeDtypeStruct((n_rows, DIM), jnp.float32),
    )(ids, table)


@jax.jit
def _lookup(ids_flat, table):
    out_tc = _tc_gather(ids_flat[N_SC:], table, N_TC)
    out_sc = _sc_gather(ids_flat[:N_SC], table, TOTAL)
    return lax.dynamic_update_slice(out_sc, out_tc, (N_SC, 0))


def kernel(input_ids, token_embeddings):
    ids_flat = input_ids.reshape(-1).astype(jnp.int32)
    out = _lookup(ids_flat, token_embeddings)
    return out.reshape(input_ids.shape + (token_embeddings.shape[1],))


# speedup vs baseline: 11.0259x; 1.0646x over previous
import functools

import jax
import jax.numpy as jnp
from jax import lax
from jax.experimental import pallas as pl
from jax.experimental.pallas import tpu as pltpu
from jax.experimental.pallas import tpu_sc as plsc

VOCAB = 100000
DIM = 2048
TOTAL = 4 * 4096

NC, NS = 2, 16
NW = NC * NS
BPW = TOTAL // NW
CHUNK = 16
NBUF = 2
NCH = BPW // CHUNK
NGRP = NCH // NBUF


def _emb_kernel(ids_hbm, table_hbm, out_hbm, idx_v, bufs, *sems):
    gsems = list(sems[:NBUF])
    ssems = list(sems[NBUF:])
    wid = lax.axis_index("s") * NC + lax.axis_index("c")
    base = wid * BPW

    pltpu.sync_copy(ids_hbm.at[pl.ds(base, BPW)], idx_v)

    def start_gather(j, b):
        off = pl.multiple_of(j * CHUNK, 8)
        pltpu.async_copy(
            table_hbm.at[idx_v.at[pl.ds(off, CHUNK)]], bufs.at[b], gsems[b])

    def wait_gather(b):
        pltpu.make_async_copy(
            table_hbm.at[pl.ds(0, CHUNK)], bufs.at[b], gsems[b]).wait()

    def start_scatter(j, b):
        row = pl.multiple_of(base + j * CHUNK, 8)
        pltpu.async_copy(bufs.at[b], out_hbm.at[pl.ds(row, CHUNK)], ssems[b])

    def wait_scatter(b):
        pltpu.make_async_copy(
            bufs.at[b], out_hbm.at[pl.ds(0, CHUNK)], ssems[b]).wait()

    for b in range(NBUF):
        start_gather(b, b)

    def group(g, carry):
        for b in range(NBUF):
            j = g * NBUF + b
            wait_gather(b)
            start_scatter(j, b)
            wait_scatter(b)
            start_gather(j + NBUF, b)
        return carry

    lax.fori_loop(0, NGRP - 1, group, 0)

    for b in range(NBUF):
        j = (NGRP - 1) * NBUF + b
        wait_gather(b)
        start_scatter(j, b)
        wait_scatter(b)


@jax.jit
def _lookup(ids_flat, table):
    mesh = plsc.VectorSubcoreMesh(core_axis_name="c", subcore_axis_name="s")
    fn = functools.partial(
        pl.kernel,
        out_type=jax.ShapeDtypeStruct((TOTAL, DIM), jnp.float32),
        mesh=mesh,
        scratch_types=[
            pltpu.VMEM((BPW,), jnp.int32),
            pltpu.VMEM((NBUF, CHUNK, DIM), jnp.float32),
        ] + [pltpu.SemaphoreType.DMA] * (2 * NBUF),
    )(_emb_kernel)
    return fn(ids_flat, table)


def kernel(input_ids, token_embeddings):
    ids_flat = input_ids.reshape(-1).astype(jnp.int32)
    out = _lookup(ids_flat, token_embeddings)
    return out.reshape(input_ids.shape + (token_embeddings.shape[1],))
